# Initial kernel scaffold; baseline (speedup 1.0000x reference)
import functools

import jax
import jax.numpy as jnp
from jax import lax
from jax.experimental import pallas as pl
from jax.experimental.pallas import tpu as pltpu
from jax.experimental.pallas import tpu_sc as plsc

N = 10000
NPAD = 10240
E = 320000
D_IN = 128
D_H = 64
D_OUT = 32
G = 64

NC = 2
NS = 16
NW = NC * NS
CH = 128
E2 = 327680
EPT2 = E2 // NW
NCH = EPT2 // CH
PADNODE = 10016
RZ = NPAD // NS

_MESH = dict(core_axis_name="c", subcore_axis_name="s", num_cores=NC,
             num_subcores=NS)


def _make_deg_kernel(interpret=False):
  mesh = plsc.VectorSubcoreMesh(**_MESH)

  @functools.partial(
      pl.kernel,
      out_type=jax.ShapeDtypeStruct((NC, NPAD, 16), jnp.float32),
      mesh=mesh,
      interpret=interpret,
      scratch_types=[
          pltpu.VMEM((CH, 16), jnp.float32),
          pltpu.VMEM((CH,), jnp.int32),
          pltpu.VMEM_SHARED((NPAD, 16), jnp.float32),
      ],
  )
  def deg_kernel(dst_hbm, ones_hbm, zeros_hbm, out_hbm, ones_v, didx, acc):
    c = lax.axis_index("c")
    s = lax.axis_index("s")
    wid = s * NC + c
    pltpu.sync_copy(zeros_hbm.at[pl.ds(0, RZ)], acc.at[pl.ds(s * RZ, RZ)])
    pltpu.sync_copy(ones_hbm, ones_v)
    plsc.subcore_barrier()

    def body(k, carry):
      off = wid * EPT2 + k * CH
      pltpu.sync_copy(dst_hbm.at[pl.ds(off, CH)], didx)
      pltpu.sync_copy(ones_v, acc.at[didx], add=True)
      return carry

    lax.fori_loop(0, NCH, body, 0)
    plsc.subcore_barrier()
    pltpu.sync_copy(acc.at[pl.ds(s * RZ, RZ)],
                    out_hbm.at[c, pl.ds(s * RZ, RZ)])

  return deg_kernel


def _make_scatter_kernel(D, interpret=False):
  mesh = plsc.VectorSubcoreMesh(**_MESH)

  @functools.partial(
      pl.kernel,
      out_type=jax.ShapeDtypeStruct((NC, NPAD, D), jnp.float32),
      mesh=mesh,
      interpret=interpret,
      scratch_types=[
          pltpu.VMEM((CH,), jnp.int32),
          pltpu.VMEM((CH,), jnp.int32),
          pltpu.VMEM((CH, D), jnp.float32),
          pltpu.VMEM_SHARED((NPAD, D), jnp.float32),
          pltpu.SemaphoreType.DMA,
      ],
  )
  def scat_kernel(g_hbm, src_hbm, dst_hbm, zeros_hbm, out_hbm,
                  sidx, didx, rows, acc, gsem):
    c = lax.axis_index("c")
    s = lax.axis_index("s")
    wid = s * NC + c
    pltpu.sync_copy(zeros_hbm.at[pl.ds(0, RZ)], acc.at[pl.ds(s * RZ, RZ)])
    plsc.subcore_barrier()

    def body(k, carry):
      off = wid * EPT2 + k * CH
      pltpu.sync_copy(src_hbm.at[pl.ds(off, CH)], sidx)
      pltpu.sync_copy(dst_hbm.at[pl.ds(off, CH)], didx)
      pltpu.async_copy(g_hbm.at[sidx], rows, gsem).wait()
      pltpu.sync_copy(rows, acc.at[didx], add=True)
      return carry

    lax.fori_loop(0, NCH, body, 0)
    plsc.subcore_barrier()
    pltpu.sync_copy(acc.at[pl.ds(s * RZ, RZ)],
                    out_hbm.at[c, pl.ds(s * RZ, RZ)])

  return scat_kernel


_R = 1024
_NBLK = NPAD // _R
_RP = 1000
_NBLKP = N // _RP


def _tc1_body(x_ref, w_ref, d0_ref, d1_ref, g_ref, dinv_ref):
  deg = d0_ref[...] + d1_ref[...] + 1.0
  dinv = lax.rsqrt(deg)
  h = jnp.dot(x_ref[...], w_ref[...], preferred_element_type=jnp.float32)
  g_ref[...] = h * dinv[:, 0:1]
  dinv_ref[...] = dinv


def _tc1(x_pad, W1, d0, d1, interpret=False):
  return pl.pallas_call(
      _tc1_body,
      grid=(_NBLK,),
      in_specs=[
          pl.BlockSpec((_R, D_IN), lambda i: (i, 0)),
          pl.BlockSpec((D_IN, D_H), lambda i: (0, 0)),
          pl.BlockSpec((_R, 16), lambda i: (i, 0)),
          pl.BlockSpec((_R, 16), lambda i: (i, 0)),
      ],
      out_specs=[
          pl.BlockSpec((_R, D_H), lambda i: (i, 0)),
          pl.BlockSpec((_R, 16), lambda i: (i, 0)),
      ],
      out_shape=[
          jax.ShapeDtypeStruct((NPAD, D_H), jnp.float32),
          jax.ShapeDtypeStruct((NPAD, 16), jnp.float32),
      ],
      interpret=interpret,
  )(x_pad, W1, d0, d1)


def _tc2_body(p0_ref, p1_ref, g1_ref, dinv_ref, b1_ref, w2_ref, g2_ref):
  dinv = dinv_ref[:, 0:1]
  t = (p0_ref[...] + p1_ref[...] + g1_ref[...]) * dinv + b1_ref[...]
  r = jnp.maximum(t, 0.0)
  h2 = jnp.dot(r, w2_ref[...], preferred_element_type=jnp.float32)
  g2_ref[...] = h2 * dinv


def _tc2(p0, p1, g1, dinv16, b1_2d, W2, interpret=False):
  return pl.pallas_call(
      _tc2_body,
      grid=(_NBLK,),
      in_specs=[
          pl.BlockSpec((_R, D_H), lambda i: (i, 0)),
          pl.BlockSpec((_R, D_H), lambda i: (i, 0)),
          pl.BlockSpec((_R, D_H), lambda i: (i, 0)),
          pl.BlockSpec((_R, 16), lambda i: (i, 0)),
          pl.BlockSpec((1, D_H), lambda i: (0, 0)),
          pl.BlockSpec((D_H, D_OUT), lambda i: (0, 0)),
      ],
      out_specs=pl.BlockSpec((_R, D_OUT), lambda i: (i, 0)),
      out_shape=jax.ShapeDtypeStruct((NPAD, D_OUT), jnp.float32),
      interpret=interpret,
  )(p0, p1, g1, dinv16, b1_2d, W2)


def _tc3_body(q0_ref, q1_ref, g2_ref, dinv_ref, b2_ref, info_ref, out_ref,
              sacc, cacc):
  i = pl.program_id(0)

  @pl.when(i == 0)
  def _():
    sacc[...] = jnp.zeros_like(sacc)
    cacc[...] = jnp.zeros_like(cacc)

  dinv = dinv_ref[:, 0:1]
  out2 = (q0_ref[...] + q1_ref[...] + g2_ref[...]) * dinv + b2_ref[...]
  gids = lax.broadcasted_iota(jnp.int32, (G, _RP), 0)
  onehot = (gids == info_ref[0]).astype(jnp.float32)
  sacc[...] += jnp.dot(onehot, out2, preferred_element_type=jnp.float32)
  cacc[...] = cacc[...] + jnp.sum(onehot, axis=1, keepdims=True)

  @pl.when(i == _NBLKP - 1)
  def _():
    out_ref[...] = sacc[...] / jnp.maximum(cacc[:, 0:1], 1.0)


def _tc3(q0, q1, g2, dinv16, b2_2d, info3, interpret=False):
  return pl.pallas_call(
      _tc3_body,
      grid=(_NBLKP,),
      in_specs=[
          pl.BlockSpec((_RP, D_OUT), lambda i: (i, 0)),
          pl.BlockSpec((_RP, D_OUT), lambda i: (i, 0)),
          pl.BlockSpec((_RP, D_OUT), lambda i: (i, 0)),
          pl.BlockSpec((_RP, 16), lambda i: (i, 0)),
          pl.BlockSpec((1, D_OUT), lambda i: (0, 0)),
          pl.BlockSpec((1, 1, _RP), lambda i: (i, 0, 0)),
      ],
      out_specs=pl.BlockSpec((G, D_OUT), lambda i: (0, 0)),
      out_shape=jax.ShapeDtypeStruct((G, D_OUT), jnp.float32),
      scratch_shapes=[
          pltpu.VMEM((G, D_OUT), jnp.float32),
          pltpu.VMEM((G, 128), jnp.float32),
      ],
      interpret=interpret,
  )(q0, q1, g2, dinv16, b2_2d, info3)


def kernel(x, edge_index, info_batch, W1, b1, W2, b2):
  pad = jnp.full((E2 - E,), PADNODE, dtype=jnp.int32)
  srcp = jnp.concatenate([edge_index[0], pad])
  dstp = jnp.concatenate([edge_index[1], pad])
  x_pad = jnp.concatenate(
      [x, jnp.zeros((NPAD - N, D_IN), dtype=x.dtype)], axis=0)

  ones16 = jnp.ones((CH, 16), dtype=jnp.float32)
  zeros16 = jnp.zeros((RZ, 16), dtype=jnp.float32)
  zeros_h = jnp.zeros((RZ, D_H), dtype=jnp.float32)
  zeros_o = jnp.zeros((RZ, D_OUT), dtype=jnp.float32)

  degp = _make_deg_kernel()(dstp, ones16, zeros16)
  g1, dinv16 = _tc1(x_pad, W1, degp[0], degp[1])
  s1 = _make_scatter_kernel(D_H)(g1, srcp, dstp, zeros_h)
  g2 = _tc2(s1[0], s1[1], g1, dinv16, b1.reshape(1, D_H), W2)
  s2 = _make_scatter_kernel(D_OUT)(g2, srcp, dstp, zeros_o)
  out = _tc3(s2[0], s2[1], g2, dinv16, b2.reshape(1, D_OUT),
             info_batch.reshape(_NBLKP, 1, _RP))
  return out


# speedup vs baseline: 11.7708x; 11.7708x over previous
import functools

import jax
import jax.numpy as jnp
from jax import lax
from jax.experimental import pallas as pl
from jax.experimental.pallas import tpu as pltpu
from jax.experimental.pallas import tpu_sc as plsc

N = 10000
NPAD = 10240
E = 320000
D_IN = 128
D_H = 64
D_OUT = 32
G = 64

NC = 2
NS = 16
NW = NC * NS
CH = 128
E2 = 327680
EPT2 = E2 // NW
NCH = EPT2 // CH
PADNODE = 10016
RZ = NPAD // NS

_MESH = dict(core_axis_name="c", subcore_axis_name="s", num_cores=NC,
             num_subcores=NS)


def _make_deg_kernel(interpret=False):
  mesh = plsc.VectorSubcoreMesh(**_MESH)

  @functools.partial(
      pl.kernel,
      out_type=jax.ShapeDtypeStruct((NC, NPAD, 16), jnp.float32),
      mesh=mesh,
      interpret=interpret,
      compiler_params=pltpu.CompilerParams(use_tc_tiling_on_sc=False),
      scratch_types=[
          pltpu.VMEM((CH, 16), jnp.float32),
          pltpu.VMEM((CH,), jnp.int32),
          pltpu.VMEM_SHARED((NPAD, 16), jnp.float32),
      ],
  )
  def deg_kernel(dst_hbm, ones_hbm, zeros_hbm, out_hbm, ones_v, didx, acc):
    c = lax.axis_index("c")
    s = lax.axis_index("s")
    wid = s * NC + c
    pltpu.sync_copy(zeros_hbm.at[pl.ds(0, RZ)], acc.at[pl.ds(s * RZ, RZ)])
    pltpu.sync_copy(ones_hbm, ones_v)
    plsc.subcore_barrier()

    def body(k, carry):
      off = wid * EPT2 + k * CH
      pltpu.sync_copy(dst_hbm.at[pl.ds(off, CH)], didx)
      pltpu.sync_copy(ones_v, acc.at[didx], add=True)
      return carry

    lax.fori_loop(0, NCH, body, 0)
    plsc.subcore_barrier()
    pltpu.sync_copy(acc.at[pl.ds(s * RZ, RZ)],
                    out_hbm.at[c, pl.ds(s * RZ, RZ)])

  return deg_kernel


def _make_scatter_kernel(D, interpret=False):
  mesh = plsc.VectorSubcoreMesh(**_MESH)

  @functools.partial(
      pl.kernel,
      out_type=jax.ShapeDtypeStruct((NC, NPAD, D), jnp.float32),
      mesh=mesh,
      interpret=interpret,
      compiler_params=pltpu.CompilerParams(use_tc_tiling_on_sc=False),
      scratch_types=[
          pltpu.VMEM((CH,), jnp.int32),
          pltpu.VMEM((CH,), jnp.int32),
          pltpu.VMEM((CH, D), jnp.float32),
          pltpu.VMEM_SHARED((NPAD, D), jnp.float32),
          pltpu.SemaphoreType.DMA,
      ],
  )
  def scat_kernel(g_hbm, src_hbm, dst_hbm, zeros_hbm, out_hbm,
                  sidx, didx, rows, acc, gsem):
    c = lax.axis_index("c")
    s = lax.axis_index("s")
    wid = s * NC + c
    pltpu.sync_copy(zeros_hbm.at[pl.ds(0, RZ)], acc.at[pl.ds(s * RZ, RZ)])
    plsc.subcore_barrier()

    def body(k, carry):
      off = wid * EPT2 + k * CH
      pltpu.sync_copy(src_hbm.at[pl.ds(off, CH)], sidx)
      pltpu.sync_copy(dst_hbm.at[pl.ds(off, CH)], didx)
      pltpu.async_copy(g_hbm.at[sidx], rows, gsem).wait()
      pltpu.sync_copy(rows, acc.at[didx], add=True)
      return carry

    lax.fori_loop(0, NCH, body, 0)
    plsc.subcore_barrier()
    pltpu.sync_copy(acc.at[pl.ds(s * RZ, RZ)],
                    out_hbm.at[c, pl.ds(s * RZ, RZ)])

  return scat_kernel


_R = 1024
_NBLK = NPAD // _R
_RP = 1000
_NBLKP = N // _RP


def _tc1_body(x_ref, w_ref, d0_ref, d1_ref, g_ref, dinv_ref):
  deg = d0_ref[...] + d1_ref[...] + 1.0
  dinv = lax.rsqrt(deg)
  h = jnp.dot(x_ref[...], w_ref[...], preferred_element_type=jnp.float32)
  g_ref[...] = h * dinv[:, 0:1]
  dinv_ref[...] = dinv


def _tc1(x_pad, W1, d0, d1, interpret=False):
  return pl.pallas_call(
      _tc1_body,
      grid=(_NBLK,),
      in_specs=[
          pl.BlockSpec((_R, D_IN), lambda i: (i, 0)),
          pl.BlockSpec((D_IN, D_H), lambda i: (0, 0)),
          pl.BlockSpec((_R, 16), lambda i: (i, 0)),
          pl.BlockSpec((_R, 16), lambda i: (i, 0)),
      ],
      out_specs=[
          pl.BlockSpec((_R, D_H), lambda i: (i, 0)),
          pl.BlockSpec((_R, 16), lambda i: (i, 0)),
      ],
      out_shape=[
          jax.ShapeDtypeStruct((NPAD, D_H), jnp.float32),
          jax.ShapeDtypeStruct((NPAD, 16), jnp.float32),
      ],
      interpret=interpret,
  )(x_pad, W1, d0, d1)


def _tc2_body(p0_ref, p1_ref, g1_ref, dinv_ref, b1_ref, w2_ref, g2_ref):
  dinv = dinv_ref[:, 0:1]
  t = (p0_ref[...] + p1_ref[...] + g1_ref[...]) * dinv + b1_ref[...]
  r = jnp.maximum(t, 0.0)
  h2 = jnp.dot(r, w2_ref[...], preferred_element_type=jnp.float32)
  g2_ref[...] = h2 * dinv


def _tc2(p0, p1, g1, dinv16, b1_2d, W2, interpret=False):
  return pl.pallas_call(
      _tc2_body,
      grid=(_NBLK,),
      in_specs=[
          pl.BlockSpec((_R, D_H), lambda i: (i, 0)),
          pl.BlockSpec((_R, D_H), lambda i: (i, 0)),
          pl.BlockSpec((_R, D_H), lambda i: (i, 0)),
          pl.BlockSpec((_R, 16), lambda i: (i, 0)),
          pl.BlockSpec((1, D_H), lambda i: (0, 0)),
          pl.BlockSpec((D_H, D_OUT), lambda i: (0, 0)),
      ],
      out_specs=pl.BlockSpec((_R, D_OUT), lambda i: (i, 0)),
      out_shape=jax.ShapeDtypeStruct((NPAD, D_OUT), jnp.float32),
      interpret=interpret,
  )(p0, p1, g1, dinv16, b1_2d, W2)


def _tc3_body(q0_ref, q1_ref, g2_ref, dinv_ref, b2_ref, info_ref, out_ref,
              sacc, cacc):
  i = pl.program_id(0)

  @pl.when(i == 0)
  def _():
    sacc[...] = jnp.zeros_like(sacc)
    cacc[...] = jnp.zeros_like(cacc)

  dinv = dinv_ref[:, 0:1]
  out2 = (q0_ref[...] + q1_ref[...] + g2_ref[...]) * dinv + b2_ref[...]
  gids = lax.broadcasted_iota(jnp.int32, (G, _RP), 0)
  onehot = (gids == info_ref[0]).astype(jnp.float32)
  sacc[...] += jnp.dot(onehot, out2, preferred_element_type=jnp.float32)
  cacc[...] = cacc[...] + jnp.sum(onehot, axis=1, keepdims=True)

  @pl.when(i == _NBLKP - 1)
  def _():
    out_ref[...] = sacc[...] / jnp.maximum(cacc[:, 0:1], 1.0)


def _tc3(q0, q1, g2, dinv16, b2_2d, info3, interpret=False):
  return pl.pallas_call(
      _tc3_body,
      grid=(_NBLKP,),
      in_specs=[
          pl.BlockSpec((_RP, D_OUT), lambda i: (i, 0)),
          pl.BlockSpec((_RP, D_OUT), lambda i: (i, 0)),
          pl.BlockSpec((_RP, D_OUT), lambda i: (i, 0)),
          pl.BlockSpec((_RP, 16), lambda i: (i, 0)),
          pl.BlockSpec((1, D_OUT), lambda i: (0, 0)),
          pl.BlockSpec((1, 1, _RP), lambda i: (i, 0, 0)),
      ],
      out_specs=pl.BlockSpec((G, D_OUT), lambda i: (0, 0)),
      out_shape=jax.ShapeDtypeStruct((G, D_OUT), jnp.float32),
      scratch_shapes=[
          pltpu.VMEM((G, D_OUT), jnp.float32),
          pltpu.VMEM((G, 128), jnp.float32),
      ],
      interpret=interpret,
  )(q0, q1, g2, dinv16, b2_2d, info3)


def kernel(x, edge_index, info_batch, W1, b1, W2, b2):
  pad = jnp.full((E2 - E,), PADNODE, dtype=jnp.int32)
  srcp = jnp.concatenate([edge_index[0], pad])
  dstp = jnp.concatenate([edge_index[1], pad])
  x_pad = jnp.concatenate(
      [x, jnp.zeros((NPAD - N, D_IN), dtype=x.dtype)], axis=0)

  ones16 = jnp.ones((CH, 16), dtype=jnp.float32)
  zeros16 = jnp.zeros((RZ, 16), dtype=jnp.float32)
  zeros_h = jnp.zeros((RZ, D_H), dtype=jnp.float32)
  zeros_o = jnp.zeros((RZ, D_OUT), dtype=jnp.float32)

  degp = _make_deg_kernel()(dstp, ones16, zeros16)
  g1, dinv16 = _tc1(x_pad, W1, degp[0], degp[1])
  s1 = _make_scatter_kernel(D_H)(g1, srcp, dstp, zeros_h)
  g2 = _tc2(s1[0], s1[1], g1, dinv16, b1.reshape(1, D_H), W2)
  s2 = _make_scatter_kernel(D_OUT)(g2, srcp, dstp, zeros_o)
  out = _tc3(s2[0], s2[1], g2, dinv16, b2.reshape(1, D_OUT),
             info_batch.reshape(_NBLKP, 1, _RP))
  return out


# speedup vs baseline: 17.3260x; 1.4719x over previous
import functools

import jax
import jax.numpy as jnp
from jax import lax
from jax.experimental import pallas as pl
from jax.experimental.pallas import tpu as pltpu
from jax.experimental.pallas import tpu_sc as plsc

N = 10000
NPAD = 10240
E = 320000
D_IN = 128
D_H = 64
D_OUT = 32
G = 64

NC = 2
NS = 16
NW = NC * NS
CH = 128
E2 = 327680
EPT2 = E2 // NW
NCH = EPT2 // CH
PADNODE = 10016
RZ = NPAD // NS

_MESH = dict(core_axis_name="c", subcore_axis_name="s", num_cores=NC,
             num_subcores=NS)


def _make_deg_kernel(interpret=False):
  mesh = plsc.VectorSubcoreMesh(**_MESH)

  @functools.partial(
      pl.kernel,
      out_type=jax.ShapeDtypeStruct((NC, NPAD, 16), jnp.float32),
      mesh=mesh,
      interpret=interpret,
      compiler_params=pltpu.CompilerParams(use_tc_tiling_on_sc=False),
      scratch_types=[
          pltpu.VMEM((CH, 16), jnp.float32),
          pltpu.VMEM((NCH, CH), jnp.int32),
          pltpu.VMEM_SHARED((NPAD, 16), jnp.float32),
          pltpu.SemaphoreType.DMA,
      ],
  )
  def deg_kernel(dst_hbm, ones_hbm, zeros_hbm, out_hbm, ones_v, didx, acc,
                 dsem):
    c = lax.axis_index("c")
    s = lax.axis_index("s")
    wid = s * NC + c
    pltpu.sync_copy(zeros_hbm.at[pl.ds(0, RZ)], acc.at[pl.ds(s * RZ, RZ)])
    pltpu.sync_copy(ones_hbm, ones_v)
    pltpu.sync_copy(dst_hbm.at[wid], didx)
    plsc.subcore_barrier()

    def body(k4, carry):
      for j in range(4):
        pltpu.async_copy(ones_v, acc.at[didx.at[k4 * 4 + j]], dsem, add=True)
      for j in range(4):
        pltpu.make_async_copy(ones_v, acc.at[didx.at[k4 * 4 + j]],
                              dsem).wait()
      return carry

    lax.fori_loop(0, NCH // 4, body, 0)
    plsc.subcore_barrier()
    pltpu.sync_copy(acc.at[pl.ds(s * RZ, RZ)],
                    out_hbm.at[c, pl.ds(s * RZ, RZ)])

  return deg_kernel


def _make_scatter_kernel(D, interpret=False):
  mesh = plsc.VectorSubcoreMesh(**_MESH)

  @functools.partial(
      pl.kernel,
      out_type=jax.ShapeDtypeStruct((NC, NPAD, D), jnp.float32),
      mesh=mesh,
      interpret=interpret,
      compiler_params=pltpu.CompilerParams(use_tc_tiling_on_sc=False),
      scratch_types=[
          pltpu.VMEM((EPT2,), jnp.int32),
          pltpu.VMEM((NCH, CH), jnp.int32),
          pltpu.VMEM((CH, D), jnp.float32),
          pltpu.VMEM((CH, D), jnp.float32),
          pltpu.VMEM_SHARED((NPAD, D), jnp.float32),
          pltpu.SemaphoreType.DMA,
          pltpu.SemaphoreType.DMA,
      ],
  )
  def scat_kernel(g_hbm, src_hbm, dst_hbm, zeros_hbm, out_hbm,
                  sidx, didx, rows0, rows1, acc, gsem0, gsem1):
    c = lax.axis_index("c")
    s = lax.axis_index("s")
    wid = s * NC + c
    rows = (rows0, rows1)
    gsem = (gsem0, gsem1)
    pltpu.sync_copy(src_hbm.at[pl.ds(wid * EPT2, EPT2)], sidx)
    pltpu.sync_copy(dst_hbm.at[wid], didx)
    pltpu.sync_copy(zeros_hbm.at[pl.ds(0, RZ)], acc.at[pl.ds(s * RZ, RZ)])
    plsc.subcore_barrier()

    def gather(k, j):
      pltpu.async_copy(g_hbm.at[sidx.at[pl.ds(k * CH, CH)]], rows[j], gsem[j])

    def consume(k, j, prefetch):
      pltpu.make_async_copy(g_hbm.at[sidx.at[pl.ds(0, CH)]], rows[j],
                            gsem[j]).wait()
      pltpu.sync_copy(rows[j], acc.at[didx.at[k]], add=True)
      if prefetch:
        gather(k + 2, j)

    gather(0, 0)
    gather(1, 1)

    def body(k2, carry):
      consume(k2 * 2, 0, True)
      consume(k2 * 2 + 1, 1, True)
      return carry

    lax.fori_loop(0, NCH // 2 - 1, body, 0)
    consume(NCH - 2, 0, False)
    consume(NCH - 1, 1, False)
    plsc.subcore_barrier()
    pltpu.sync_copy(acc.at[pl.ds(s * RZ, RZ)],
                    out_hbm.at[c, pl.ds(s * RZ, RZ)])

  return scat_kernel


_R = 1024
_NBLK = NPAD // _R
_RP = 1000
_NBLKP = N // _RP


def _tc1_body(x_ref, w_ref, d0_ref, d1_ref, g_ref, dinv_ref):
  deg = d0_ref[...] + d1_ref[...] + 1.0
  dinv = lax.rsqrt(deg)
  h = jnp.dot(x_ref[...], w_ref[...], preferred_element_type=jnp.float32)
  g_ref[...] = h * dinv[:, 0:1]
  dinv_ref[...] = dinv


def _tc1(x_pad, W1, d0, d1, interpret=False):
  return pl.pallas_call(
      _tc1_body,
      grid=(_NBLK,),
      in_specs=[
          pl.BlockSpec((_R, D_IN), lambda i: (i, 0)),
          pl.BlockSpec((D_IN, D_H), lambda i: (0, 0)),
          pl.BlockSpec((_R, 16), lambda i: (i, 0)),
          pl.BlockSpec((_R, 16), lambda i: (i, 0)),
      ],
      out_specs=[
          pl.BlockSpec((_R, D_H), lambda i: (i, 0)),
          pl.BlockSpec((_R, 16), lambda i: (i, 0)),
      ],
      out_shape=[
          jax.ShapeDtypeStruct((NPAD, D_H), jnp.float32),
          jax.ShapeDtypeStruct((NPAD, 16), jnp.float32),
      ],
      interpret=interpret,
  )(x_pad, W1, d0, d1)


def _tc2_body(p0_ref, p1_ref, g1_ref, dinv_ref, b1_ref, w2_ref, g2_ref):
  dinv = dinv_ref[:, 0:1]
  t = (p0_ref[...] + p1_ref[...] + g1_ref[...]) * dinv + b1_ref[...]
  r = jnp.maximum(t, 0.0)
  h2 = jnp.dot(r, w2_ref[...], preferred_element_type=jnp.float32)
  g2_ref[...] = h2 * dinv


def _tc2(p0, p1, g1, dinv16, b1_2d, W2, interpret=False):
  return pl.pallas_call(
      _tc2_body,
      grid=(_NBLK,),
      in_specs=[
          pl.BlockSpec((_R, D_H), lambda i: (i, 0)),
          pl.BlockSpec((_R, D_H), lambda i: (i, 0)),
          pl.BlockSpec((_R, D_H), lambda i: (i, 0)),
          pl.BlockSpec((_R, 16), lambda i: (i, 0)),
          pl.BlockSpec((1, D_H), lambda i: (0, 0)),
          pl.BlockSpec((D_H, D_OUT), lambda i: (0, 0)),
      ],
      out_specs=pl.BlockSpec((_R, D_OUT), lambda i: (i, 0)),
      out_shape=jax.ShapeDtypeStruct((NPAD, D_OUT), jnp.float32),
      interpret=interpret,
  )(p0, p1, g1, dinv16, b1_2d, W2)


def _tc3_body(q0_ref, q1_ref, g2_ref, dinv_ref, b2_ref, info_ref, out_ref,
              sacc, cacc):
  i = pl.program_id(0)

  @pl.when(i == 0)
  def _():
    sacc[...] = jnp.zeros_like(sacc)
    cacc[...] = jnp.zeros_like(cacc)

  dinv = dinv_ref[:, 0:1]
  out2 = (q0_ref[...] + q1_ref[...] + g2_ref[...]) * dinv + b2_ref[...]
  gids = lax.broadcasted_iota(jnp.int32, (G, _RP), 0)
  onehot = (gids == info_ref[0]).astype(jnp.float32)
  sacc[...] += jnp.dot(onehot, out2, preferred_element_type=jnp.float32)
  cacc[...] = cacc[...] + jnp.sum(onehot, axis=1, keepdims=True)

  @pl.when(i == _NBLKP - 1)
  def _():
    out_ref[...] = sacc[...] / jnp.maximum(cacc[:, 0:1], 1.0)


def _tc3(q0, q1, g2, dinv16, b2_2d, info3, interpret=False):
  return pl.pallas_call(
      _tc3_body,
      grid=(_NBLKP,),
      in_specs=[
          pl.BlockSpec((_RP, D_OUT), lambda i: (i, 0)),
          pl.BlockSpec((_RP, D_OUT), lambda i: (i, 0)),
          pl.BlockSpec((_RP, D_OUT), lambda i: (i, 0)),
          pl.BlockSpec((_RP, 16), lambda i: (i, 0)),
          pl.BlockSpec((1, D_OUT), lambda i: (0, 0)),
          pl.BlockSpec((1, 1, _RP), lambda i: (i, 0, 0)),
      ],
      out_specs=pl.BlockSpec((G, D_OUT), lambda i: (0, 0)),
      out_shape=jax.ShapeDtypeStruct((G, D_OUT), jnp.float32),
      scratch_shapes=[
          pltpu.VMEM((G, D_OUT), jnp.float32),
          pltpu.VMEM((G, 128), jnp.float32),
      ],
      interpret=interpret,
  )(q0, q1, g2, dinv16, b2_2d, info3)


def kernel(x, edge_index, info_batch, W1, b1, W2, b2):
  pad = jnp.full((E2 - E,), PADNODE, dtype=jnp.int32)
  srcp = jnp.concatenate([edge_index[0], pad])
  dstp = jnp.concatenate([edge_index[1], pad])
  x_pad = jnp.concatenate(
      [x, jnp.zeros((NPAD - N, D_IN), dtype=x.dtype)], axis=0)

  ones16 = jnp.ones((CH, 16), dtype=jnp.float32)
  zeros16 = jnp.zeros((RZ, 16), dtype=jnp.float32)
  zeros_h = jnp.zeros((RZ, D_H), dtype=jnp.float32)
  zeros_o = jnp.zeros((RZ, D_OUT), dtype=jnp.float32)

  dst3 = dstp.reshape(NW, NCH, CH)
  degp = _make_deg_kernel()(dst3, ones16, zeros16)
  g1, dinv16 = _tc1(x_pad, W1, degp[0], degp[1])
  s1 = _make_scatter_kernel(D_H)(g1, srcp, dst3, zeros_h)
  g2 = _tc2(s1[0], s1[1], g1, dinv16, b1.reshape(1, D_H), W2)
  s2 = _make_scatter_kernel(D_OUT)(g2, srcp, dst3, zeros_o)
  out = _tc3(s2[0], s2[1], g2, dinv16, b2.reshape(1, D_OUT),
             info_batch.reshape(_NBLKP, 1, _RP))
  return out


# speedup vs baseline: 35.5975x; 2.0546x over previous
import functools

import jax
import jax.numpy as jnp
from jax import lax
from jax.experimental import pallas as pl
from jax.experimental.pallas import tpu as pltpu
from jax.experimental.pallas import tpu_sc as plsc

N = 10000
NPAD = 10240
E = 320000
D_IN = 128
D_H = 64
D_OUT = 32
G = 64

NC = 2
NS = 16
NW = NC * NS
CH = 128
E2 = 327680
EPT2 = E2 // NW
NCH = EPT2 // CH
PADNODE = 10016
RZ = NPAD // NS

_MESH = dict(core_axis_name="c", subcore_axis_name="s", num_cores=NC,
             num_subcores=NS)


def _make_deg_kernel(interpret=False):
  mesh = plsc.VectorSubcoreMesh(**_MESH)

  @functools.partial(
      pl.kernel,
      out_type=jax.ShapeDtypeStruct((NC, NPAD, 16), jnp.float32),
      mesh=mesh,
      interpret=interpret,
      compiler_params=pltpu.CompilerParams(use_tc_tiling_on_sc=False),
      scratch_types=[
          pltpu.VMEM((CH, 16), jnp.float32),
          pltpu.VMEM((NCH, CH), jnp.int32),
          pltpu.VMEM_SHARED((NPAD, 16), jnp.float32),
          pltpu.SemaphoreType.DMA,
      ],
  )
  def deg_kernel(dst_hbm, ones_hbm, zeros_hbm, out_hbm, ones_v, didx, acc,
                 dsem):
    c = lax.axis_index("c")
    s = lax.axis_index("s")
    wid = s * NC + c
    pltpu.sync_copy(zeros_hbm.at[pl.ds(0, RZ)], acc.at[pl.ds(s * RZ, RZ)])
    pltpu.sync_copy(ones_hbm, ones_v)
    pltpu.sync_copy(dst_hbm.at[wid], didx)
    plsc.subcore_barrier()

    def body(k4, carry):
      for j in range(4):
        pltpu.async_copy(ones_v, acc.at[didx.at[k4 * 4 + j]], dsem, add=True)
      for j in range(4):
        pltpu.make_async_copy(ones_v, acc.at[didx.at[k4 * 4 + j]],
                              dsem).wait()
      return carry

    lax.fori_loop(0, NCH // 4, body, 0)
    plsc.subcore_barrier()
    pltpu.sync_copy(acc.at[pl.ds(s * RZ, RZ)],
                    out_hbm.at[c, pl.ds(s * RZ, RZ)])

  return deg_kernel


def _make_scatter_kernel(D, interpret=False):
  mesh = plsc.VectorSubcoreMesh(**_MESH)

  @functools.partial(
      pl.kernel,
      out_type=jax.ShapeDtypeStruct((NC, NPAD, D), jnp.float32),
      mesh=mesh,
      interpret=interpret,
      compiler_params=pltpu.CompilerParams(use_tc_tiling_on_sc=False),
      scratch_types=[
          pltpu.VMEM((EPT2,), jnp.int32),
          pltpu.VMEM((NCH, CH), jnp.int32),
          pltpu.VMEM((CH, D), jnp.float32),
          pltpu.VMEM((CH, D), jnp.float32),
          pltpu.VMEM_SHARED((NPAD, D), jnp.float32),
          pltpu.VMEM_SHARED((NPAD, D), jnp.float32),
          pltpu.SemaphoreType.DMA,
          pltpu.SemaphoreType.DMA,
      ],
  )
  def scat_kernel(g_hbm, src_hbm, dst_hbm, zeros_hbm, out_hbm,
                  sidx, didx, rows0, rows1, acc, g_sh, gsem0, gsem1):
    c = lax.axis_index("c")
    s = lax.axis_index("s")
    wid = s * NC + c
    rows = (rows0, rows1)
    gsem = (gsem0, gsem1)
    pltpu.sync_copy(src_hbm.at[pl.ds(wid * EPT2, EPT2)], sidx)
    pltpu.sync_copy(dst_hbm.at[wid], didx)
    pltpu.sync_copy(zeros_hbm.at[pl.ds(0, RZ)], acc.at[pl.ds(s * RZ, RZ)])
    pltpu.sync_copy(g_hbm.at[pl.ds(s * RZ, RZ)], g_sh.at[pl.ds(s * RZ, RZ)])
    plsc.subcore_barrier()

    def gather(k, j):
      pltpu.async_copy(g_sh.at[sidx.at[pl.ds(k * CH, CH)]], rows[j], gsem[j])

    def consume(k, j, prefetch):
      pltpu.make_async_copy(g_sh.at[sidx.at[pl.ds(0, CH)]], rows[j],
                            gsem[j]).wait()
      pltpu.sync_copy(rows[j], acc.at[didx.at[k]], add=True)
      if prefetch:
        gather(k + 2, j)

    gather(0, 0)
    gather(1, 1)

    def body(k2, carry):
      consume(k2 * 2, 0, True)
      consume(k2 * 2 + 1, 1, True)
      return carry

    lax.fori_loop(0, NCH // 2 - 1, body, 0)
    consume(NCH - 2, 0, False)
    consume(NCH - 1, 1, False)
    plsc.subcore_barrier()
    pltpu.sync_copy(acc.at[pl.ds(s * RZ, RZ)],
                    out_hbm.at[c, pl.ds(s * RZ, RZ)])

  return scat_kernel


_R = 1024
_NBLK = NPAD // _R
_RP = 1000
_NBLKP = N // _RP


def _tc1_body(x_ref, w_ref, d0_ref, d1_ref, g_ref, dinv_ref):
  deg = d0_ref[...] + d1_ref[...] + 1.0
  dinv = lax.rsqrt(deg)
  h = jnp.dot(x_ref[...], w_ref[...], preferred_element_type=jnp.float32)
  g_ref[...] = h * dinv[:, 0:1]
  dinv_ref[...] = dinv


def _tc1(x_pad, W1, d0, d1, interpret=False):
  return pl.pallas_call(
      _tc1_body,
      grid=(_NBLK,),
      in_specs=[
          pl.BlockSpec((_R, D_IN), lambda i: (i, 0)),
          pl.BlockSpec((D_IN, D_H), lambda i: (0, 0)),
          pl.BlockSpec((_R, 16), lambda i: (i, 0)),
          pl.BlockSpec((_R, 16), lambda i: (i, 0)),
      ],
      out_specs=[
          pl.BlockSpec((_R, D_H), lambda i: (i, 0)),
          pl.BlockSpec((_R, 16), lambda i: (i, 0)),
      ],
      out_shape=[
          jax.ShapeDtypeStruct((NPAD, D_H), jnp.float32),
          jax.ShapeDtypeStruct((NPAD, 16), jnp.float32),
      ],
      interpret=interpret,
  )(x_pad, W1, d0, d1)


def _tc2_body(p0_ref, p1_ref, g1_ref, dinv_ref, b1_ref, w2_ref, g2_ref):
  dinv = dinv_ref[:, 0:1]
  t = (p0_ref[...] + p1_ref[...] + g1_ref[...]) * dinv + b1_ref[...]
  r = jnp.maximum(t, 0.0)
  h2 = jnp.dot(r, w2_ref[...], preferred_element_type=jnp.float32)
  g2_ref[...] = h2 * dinv


def _tc2(p0, p1, g1, dinv16, b1_2d, W2, interpret=False):
  return pl.pallas_call(
      _tc2_body,
      grid=(_NBLK,),
      in_specs=[
          pl.BlockSpec((_R, D_H), lambda i: (i, 0)),
          pl.BlockSpec((_R, D_H), lambda i: (i, 0)),
          pl.BlockSpec((_R, D_H), lambda i: (i, 0)),
          pl.BlockSpec((_R, 16), lambda i: (i, 0)),
          pl.BlockSpec((1, D_H), lambda i: (0, 0)),
          pl.BlockSpec((D_H, D_OUT), lambda i: (0, 0)),
      ],
      out_specs=pl.BlockSpec((_R, D_OUT), lambda i: (i, 0)),
      out_shape=jax.ShapeDtypeStruct((NPAD, D_OUT), jnp.float32),
      interpret=interpret,
  )(p0, p1, g1, dinv16, b1_2d, W2)


def _tc3_body(q0_ref, q1_ref, g2_ref, dinv_ref, b2_ref, info_ref, out_ref,
              sacc, cacc):
  i = pl.program_id(0)

  @pl.when(i == 0)
  def _():
    sacc[...] = jnp.zeros_like(sacc)
    cacc[...] = jnp.zeros_like(cacc)

  dinv = dinv_ref[:, 0:1]
  out2 = (q0_ref[...] + q1_ref[...] + g2_ref[...]) * dinv + b2_ref[...]
  gids = lax.broadcasted_iota(jnp.int32, (G, _RP), 0)
  onehot = (gids == info_ref[0]).astype(jnp.float32)
  sacc[...] += jnp.dot(onehot, out2, preferred_element_type=jnp.float32)
  cacc[...] = cacc[...] + jnp.sum(onehot, axis=1, keepdims=True)

  @pl.when(i == _NBLKP - 1)
  def _():
    out_ref[...] = sacc[...] / jnp.maximum(cacc[:, 0:1], 1.0)


def _tc3(q0, q1, g2, dinv16, b2_2d, info3, interpret=False):
  return pl.pallas_call(
      _tc3_body,
      grid=(_NBLKP,),
      in_specs=[
          pl.BlockSpec((_RP, D_OUT), lambda i: (i, 0)),
          pl.BlockSpec((_RP, D_OUT), lambda i: (i, 0)),
          pl.BlockSpec((_RP, D_OUT), lambda i: (i, 0)),
          pl.BlockSpec((_RP, 16), lambda i: (i, 0)),
          pl.BlockSpec((1, D_OUT), lambda i: (0, 0)),
          pl.BlockSpec((1, 1, _RP), lambda i: (i, 0, 0)),
      ],
      out_specs=pl.BlockSpec((G, D_OUT), lambda i: (0, 0)),
      out_shape=jax.ShapeDtypeStruct((G, D_OUT), jnp.float32),
      scratch_shapes=[
          pltpu.VMEM((G, D_OUT), jnp.float32),
          pltpu.VMEM((G, 128), jnp.float32),
      ],
      interpret=interpret,
  )(q0, q1, g2, dinv16, b2_2d, info3)


def kernel(x, edge_index, info_batch, W1, b1, W2, b2):
  pad = jnp.full((E2 - E,), PADNODE, dtype=jnp.int32)
  srcp = jnp.concatenate([edge_index[0], pad])
  dstp = jnp.concatenate([edge_index[1], pad])
  x_pad = jnp.concatenate(
      [x, jnp.zeros((NPAD - N, D_IN), dtype=x.dtype)], axis=0)

  ones16 = jnp.ones((CH, 16), dtype=jnp.float32)
  zeros16 = jnp.zeros((RZ, 16), dtype=jnp.float32)
  zeros_h = jnp.zeros((RZ, D_H), dtype=jnp.float32)
  zeros_o = jnp.zeros((RZ, D_OUT), dtype=jnp.float32)

  dst3 = dstp.reshape(NW, NCH, CH)
  degp = _make_deg_kernel()(dst3, ones16, zeros16)
  g1, dinv16 = _tc1(x_pad, W1, degp[0], degp[1])
  s1 = _make_scatter_kernel(D_H)(g1, srcp, dst3, zeros_h)
  g2 = _tc2(s1[0], s1[1], g1, dinv16, b1.reshape(1, D_H), W2)
  s2 = _make_scatter_kernel(D_OUT)(g2, srcp, dst3, zeros_o)
  out = _tc3(s2[0], s2[1], g2, dinv16, b2.reshape(1, D_OUT),
             info_batch.reshape(_NBLKP, 1, _RP))
  return out


# speedup vs baseline: 40.6382x; 1.1416x over previous
import functools

import jax
import jax.numpy as jnp
from jax import lax
from jax.experimental import pallas as pl
from jax.experimental.pallas import tpu as pltpu
from jax.experimental.pallas import tpu_sc as plsc

N = 10000
NPAD = 10240
E = 320000
D_IN = 128
D_H = 64
D_OUT = 32
G = 64

NC = 2
NS = 16
NW = NC * NS
CH = 128
E2 = 327680
EPT2 = E2 // NW
NCH = EPT2 // CH
PADNODE = 10016
RZ = NPAD // NS

_MESH = dict(core_axis_name="c", subcore_axis_name="s", num_cores=NC,
             num_subcores=NS)


def _make_deg_kernel(interpret=False):
  mesh = plsc.VectorSubcoreMesh(**_MESH)

  @functools.partial(
      pl.kernel,
      out_type=jax.ShapeDtypeStruct((NC, NPAD, 16), jnp.float32),
      mesh=mesh,
      interpret=interpret,
      compiler_params=pltpu.CompilerParams(use_tc_tiling_on_sc=False),
      scratch_types=[
          pltpu.VMEM((CH, 16), jnp.float32),
          pltpu.VMEM((NCH, CH), jnp.int32),
          pltpu.VMEM_SHARED((NPAD, 16), jnp.float32),
          pltpu.SemaphoreType.DMA,
      ],
  )
  def deg_kernel(dst_hbm, ones_hbm, zeros_hbm, out_hbm, ones_v, didx, acc,
                 dsem):
    c = lax.axis_index("c")
    s = lax.axis_index("s")
    wid = s * NC + c
    descs = [
        pltpu.async_copy(zeros_hbm.at[pl.ds(0, RZ)],
                         acc.at[pl.ds(s * RZ, RZ)], dsem),
        pltpu.async_copy(ones_hbm, ones_v, dsem),
        pltpu.async_copy(dst_hbm.at[wid], didx, dsem),
    ]
    for d in descs:
      d.wait()
    plsc.subcore_barrier()

    def body(k4, carry):
      for j in range(4):
        pltpu.async_copy(ones_v, acc.at[didx.at[k4 * 4 + j]], dsem, add=True)
      for j in range(4):
        pltpu.make_async_copy(ones_v, acc.at[didx.at[k4 * 4 + j]],
                              dsem).wait()
      return carry

    lax.fori_loop(0, NCH // 4, body, 0)
    plsc.subcore_barrier()
    pltpu.sync_copy(acc.at[pl.ds(s * RZ, RZ)],
                    out_hbm.at[c, pl.ds(s * RZ, RZ)])

  return deg_kernel


def _make_scatter_kernel(D, interpret=False):
  mesh = plsc.VectorSubcoreMesh(**_MESH)

  @functools.partial(
      pl.kernel,
      out_type=jax.ShapeDtypeStruct((NC, NPAD, D), jnp.float32),
      mesh=mesh,
      interpret=interpret,
      compiler_params=pltpu.CompilerParams(use_tc_tiling_on_sc=False),
      scratch_types=[
          pltpu.VMEM((EPT2,), jnp.int32),
          pltpu.VMEM((NCH, CH), jnp.int32),
          pltpu.VMEM((CH, D), jnp.float32),
          pltpu.VMEM((CH, D), jnp.float32),
          pltpu.VMEM_SHARED((NPAD, D), jnp.float32),
          pltpu.VMEM_SHARED((NPAD, D), jnp.float32),
          pltpu.SemaphoreType.DMA,
          pltpu.SemaphoreType.DMA,
      ],
  )
  def scat_kernel(g_hbm, src_hbm, dst_hbm, zeros_hbm, out_hbm,
                  sidx, didx, rows0, rows1, acc, g_sh, gsem0, gsem1):
    c = lax.axis_index("c")
    s = lax.axis_index("s")
    wid = s * NC + c
    rows = (rows0, rows1)
    gsem = (gsem0, gsem1)
    prologue = (
        lambda sem: pltpu.async_copy(src_hbm.at[pl.ds(wid * EPT2, EPT2)],
                                     sidx, sem),
        lambda sem: pltpu.async_copy(dst_hbm.at[wid], didx, sem),
        lambda sem: pltpu.async_copy(zeros_hbm.at[pl.ds(0, RZ)],
                                     acc.at[pl.ds(s * RZ, RZ)], sem),
        lambda sem: pltpu.async_copy(g_hbm.at[pl.ds(s * RZ, RZ)],
                                     g_sh.at[pl.ds(s * RZ, RZ)], sem),
    )
    descs = [issue(gsem0) for issue in prologue]
    for d in descs:
      d.wait()
    plsc.subcore_barrier()

    def gather(k, j):
      pltpu.async_copy(g_sh.at[sidx.at[pl.ds(k * CH, CH)]], rows[j], gsem[j])

    def consume(k, j, prefetch):
      pltpu.make_async_copy(g_sh.at[sidx.at[pl.ds(0, CH)]], rows[j],
                            gsem[j]).wait()
      pltpu.sync_copy(rows[j], acc.at[didx.at[k]], add=True)
      if prefetch:
        gather(k + 2, j)

    gather(0, 0)
    gather(1, 1)

    def body(k2, carry):
      consume(k2 * 2, 0, True)
      consume(k2 * 2 + 1, 1, True)
      return carry

    lax.fori_loop(0, NCH // 2 - 1, body, 0)
    consume(NCH - 2, 0, False)
    consume(NCH - 1, 1, False)
    plsc.subcore_barrier()
    pltpu.sync_copy(acc.at[pl.ds(s * RZ, RZ)],
                    out_hbm.at[c, pl.ds(s * RZ, RZ)])

  return scat_kernel


def _tc1_body(x_ref, w_ref, d_ref, g_ref, dinv_ref):
  deg = d_ref[0] + d_ref[1] + 1.0
  dinv = lax.rsqrt(deg)
  h = jnp.dot(x_ref[...], w_ref[...], preferred_element_type=jnp.float32)
  g_ref[0:N, :] = h * dinv[0:N, 0:1]
  g_ref[N:NPAD, :] = jnp.zeros((NPAD - N, D_H), jnp.float32)
  dinv_ref[...] = dinv


def _tc1(x, W1, degp, interpret=False):
  return pl.pallas_call(
      _tc1_body,
      out_shape=[
          jax.ShapeDtypeStruct((NPAD, D_H), jnp.float32),
          jax.ShapeDtypeStruct((NPAD, 16), jnp.float32),
      ],
      interpret=interpret,
  )(x, W1, degp)


def _tc2_body(s_ref, g1_ref, dinv_ref, b1_ref, w2_ref, g2_ref):
  dinv = dinv_ref[:, 0:1]
  t = (s_ref[0] + s_ref[1] + g1_ref[...]) * dinv + b1_ref[...]
  r = jnp.maximum(t, 0.0)
  h2 = jnp.dot(r, w2_ref[...], preferred_element_type=jnp.float32)
  g2_ref[...] = h2 * dinv


def _tc2(s1, g1, dinv16, b1_2d, W2, interpret=False):
  return pl.pallas_call(
      _tc2_body,
      out_shape=jax.ShapeDtypeStruct((NPAD, D_OUT), jnp.float32),
      interpret=interpret,
  )(s1, g1, dinv16, b1_2d, W2)


def _tc3_body(s_ref, g2_ref, dinv_ref, b2_ref, info_ref, out_ref):
  dinv = dinv_ref[0:N, 0:1]
  out2 = (s_ref[0, 0:N] + s_ref[1, 0:N] + g2_ref[0:N]) * dinv + b2_ref[...]
  gids = lax.broadcasted_iota(jnp.int32, (G, N), 0).astype(jnp.float32)
  onehot = (gids == info_ref[...]).astype(jnp.float32)
  sums = jnp.dot(onehot, out2, preferred_element_type=jnp.float32)
  counts = jnp.sum(onehot, axis=1, keepdims=True)
  out_ref[...] = sums / jnp.maximum(counts, 1.0)


def _tc3(s2, g2, dinv16, b2_2d, info_f, interpret=False):
  return pl.pallas_call(
      _tc3_body,
      out_shape=jax.ShapeDtypeStruct((G, D_OUT), jnp.float32),
      interpret=interpret,
  )(s2, g2, dinv16, b2_2d, info_f)


def kernel(x, edge_index, info_batch, W1, b1, W2, b2):
  pad = jnp.full((E2 - E,), PADNODE, dtype=jnp.int32)
  srcp = jnp.concatenate([edge_index[0], pad])
  dst3 = jnp.concatenate([edge_index[1], pad]).reshape(NW, NCH, CH)

  ones16 = jnp.ones((CH, 16), dtype=jnp.float32)
  zeros16 = jnp.zeros((RZ, 16), dtype=jnp.float32)
  zeros_h = jnp.zeros((RZ, D_H), dtype=jnp.float32)
  zeros_o = jnp.zeros((RZ, D_OUT), dtype=jnp.float32)

  degp = _make_deg_kernel()(dst3, ones16, zeros16)
  g1, dinv16 = _tc1(x, W1, degp)
  s1 = _make_scatter_kernel(D_H)(g1, srcp, dst3, zeros_h)
  g2 = _tc2(s1, g1, dinv16, b1.reshape(1, D_H), W2)
  s2 = _make_scatter_kernel(D_OUT)(g2, srcp, dst3, zeros_o)
  out = _tc3(s2, g2, dinv16, b2.reshape(1, D_OUT),
             info_batch.astype(jnp.float32).reshape(1, N))
  return out
